# baseline (speedup 1.0000x reference)
import functools

import jax
import jax.numpy as jnp
from jax import lax
from jax.experimental import pallas as pl
from jax.experimental.pallas import tpu as pltpu
from jax.experimental.pallas import tpu_sc as plsc

B = 16384
N_FIELDS = 26
FIELD_VOCAB = 100000
EMB = 16
N_SLOTS = N_FIELDS + 2

_INFO = plsc.get_sparse_core_info()
NC = _INFO.num_cores
NS = _INFO.num_subcores
NW = NC * NS

ROWS_UI = B // NW
ROWS_SP = N_FIELDS * B // NW
CH = 512
SP_CHUNKS = ROWS_SP // CH


def _gather_body(uid, iid, spidx, ut, it, st, out, idx_v, rows_v, sem):
  wid = lax.axis_index("s") * NC + lax.axis_index("c")

  def one(idx_hbm, idx_off, table, out_off):
    pltpu.sync_copy(idx_hbm.at[pl.ds(idx_off, CH)], idx_v)
    pltpu.async_copy(table.at[idx_v], rows_v, sem).wait()
    pltpu.sync_copy(rows_v, out.at[pl.ds(out_off, CH)])

  base = wid * ROWS_UI
  one(uid, base, ut, base)
  one(iid, base, it, B + base)

  def body(j, carry):
    off = wid * ROWS_SP + j * CH
    one(spidx, off, st, 2 * B + off)
    return carry

  lax.fori_loop(0, SP_CHUNKS, body, 0)


_gather = pl.kernel(
    _gather_body,
    out_type=jax.ShapeDtypeStruct((N_SLOTS * B, EMB), jnp.float32),
    mesh=plsc.VectorSubcoreMesh(core_axis_name="c", subcore_axis_name="s"),
    scratch_types=[
        pltpu.VMEM((CH,), jnp.int32),
        pltpu.VMEM((CH, EMB), jnp.float32),
        pltpu.SemaphoreType.DMA,
    ],
    compiler_params=pltpu.CompilerParams(use_tc_tiling_on_sc=False),
)

BB = 512
GRID = B // BB


def _head_body(x_ref, w1_ref, b1_ref, w2_ref, b2_ref, w3_ref, b3_ref,
               w4_ref, b4_ref, o_ref):
  sv = jnp.zeros((BB, EMB), jnp.float32)
  qv = jnp.zeros((BB, EMB), jnp.float32)
  h0 = jnp.zeros((BB, EMB), jnp.float32)
  for f in range(N_SLOTS):
    xf = x_ref[f]
    sv = sv + xf
    qv = qv + xf * xf
    h0 = h0 + jnp.dot(xf, w1_ref[f], preferred_element_type=jnp.float32)
  s = jnp.sum(sv, axis=1, keepdims=True)
  q = jnp.sum(qv, axis=1, keepdims=True)
  fm = 0.5 * (s * s - q)

  h = jnp.maximum(h0 + b1_ref[...], 0.0)
  h = jnp.maximum(jnp.dot(h, w2_ref[...], preferred_element_type=jnp.float32)
                  + b2_ref[...], 0.0)
  h = jnp.maximum(jnp.dot(h, w3_ref[...], preferred_element_type=jnp.float32)
                  + b3_ref[...], 0.0)
  d = jnp.sum(h * w4_ref[...], axis=1, keepdims=True) + b4_ref[...]
  logit = (d + fm)[:, 0]
  pos = 1.0 / (1.0 + jnp.exp(-logit))
  neg = jnp.exp(logit) / (1.0 + jnp.exp(logit))
  o_ref[...] = jnp.where(logit >= 0.0, pos, neg)[None, :]


_full = lambda i: (0, 0)
_head = pl.pallas_call(
    _head_body,
    grid=(GRID,),
    in_specs=[
        pl.BlockSpec((N_SLOTS, BB, EMB), lambda i: (0, i, 0)),
        pl.BlockSpec((N_SLOTS, EMB, EMB), lambda i: (0, 0, 0)),
        pl.BlockSpec((1, EMB), _full),
        pl.BlockSpec((EMB, EMB), _full),
        pl.BlockSpec((1, EMB), _full),
        pl.BlockSpec((EMB, EMB), _full),
        pl.BlockSpec((1, EMB), _full),
        pl.BlockSpec((1, EMB), _full),
        pl.BlockSpec((1, 1), _full),
    ],
    out_specs=pl.BlockSpec((1, BB), lambda i: (0, i)),
    out_shape=jax.ShapeDtypeStruct((1, B), jnp.float32),
)


@jax.jit
def kernel(user_ids, item_ids, sparse_features, user_table, item_table,
           sparse_tables, W1, b1, W2, b2, W3, b3, W4, b4):
  offs = (jnp.arange(N_FIELDS, dtype=jnp.int32) * FIELD_VOCAB)[:, None]
  spidx = (sparse_features.T + offs).reshape(-1)
  spflat = sparse_tables.reshape(N_FIELDS * FIELD_VOCAB, EMB)
  comb = _gather(user_ids, item_ids, spidx, user_table, item_table, spflat)
  out = _head(comb.reshape(N_SLOTS, B, EMB),
              W1.reshape(N_SLOTS, EMB, EMB),
              b1.reshape(1, EMB),
              W2, b2.reshape(1, EMB),
              W3, b3.reshape(1, EMB),
              W4.reshape(1, EMB),
              b4.reshape(1, 1))
  return out.reshape(B)
